# baseline (speedup 1.0000x reference)
import jax
import jax.numpy as jnp
from jax import lax
from jax.experimental import pallas as pl
from jax.experimental.pallas import tpu as pltpu
from jax.experimental.pallas import tpu_sc as plsc

N_ROWS = 1000000
D_EMBED = 64
B_TOTAL = 1024 * 200
NW = 32

NBLK = 999936 // 128
TAIL_ROWS = N_ROWS - 999936
TAIL_PAIRS = TAIL_ROWS // 2

B_PER_W = B_TOTAL // NW
CHUNK = 128
NCHUNK = B_PER_W // CHUNK
NBUF = 5
LOOK = 3
NROUND = NCHUNK // NBUF
GPB = CHUNK // 16


def _transpose_kernel(embT_hbm, tail_hbm, tab_hbm, src_v, dst_v, gsem, wsem):
    wid = lax.axis_index("s") * 2 + lax.axis_index("c")
    start = 244 * wid + jnp.minimum(wid, 4)
    count = jnp.where(wid < 4, 245, 244)
    iota16 = lax.iota(jnp.int32, 16)

    def block_body(k, carry):
        ti = start + k
        c0 = ti * 128
        for td in range(8):
            pltpu.make_async_copy(
                embT_hbm.at[pl.ds(8 * td, 8), pl.ds(c0, 128)],
                src_v.at[pl.ds(8 * td, 8)],
                gsem,
            ).start()
        pltpu.make_async_copy(
            embT_hbm.at[pl.ds(0, 64), pl.ds(0, 128)], src_v, gsem
        ).wait()

        def col_body(r, c):
            half = 64 * (r % 2)
            rvec = jnp.full((16,), r, jnp.int32)
            for j in range(4):
                vals = plsc.load_gather(src_v, [16 * j + iota16, rvec])
                dst_v[r // 2, pl.ds(half + 16 * j, 16)] = vals
            return c

        lax.fori_loop(0, 128, col_body, 0)
        pltpu.sync_copy(dst_v, tab_hbm.at[pl.ds(64 * ti, 64)])
        return carry

    lax.fori_loop(0, count, block_body, 0)

    @pl.when(wid == NW - 1)
    def _tail():
        pltpu.sync_copy(tail_hbm, src_v)
        for q in range(TAIL_PAIRS):
            for j in range(4):
                dst_v[q, pl.ds(16 * j, 16)] = src_v[2 * q, pl.ds(16 * j, 16)]
                dst_v[q, pl.ds(64 + 16 * j, 16)] = src_v[
                    2 * q + 1, pl.ds(16 * j, 16)
                ]
        pltpu.sync_copy(
            dst_v.at[pl.ds(0, TAIL_PAIRS)],
            tab_hbm.at[pl.ds(NBLK * 64, TAIL_PAIRS)],
        )


def _gather_kernel(idx_hbm, table_hbm, out_hbm, idx_v, rows_v, gsem, osem):
    wid = lax.axis_index("s") * 2 + lax.axis_index("c")
    base = wid * B_PER_W
    pltpu.sync_copy(idx_hbm.at[pl.ds(base, B_PER_W)], idx_v)

    class _G:
        def __init__(self, g, b):
            self.g, self.b = g, b

        def start(self):
            for j in range(GPB):
                ivec = idx_v[pl.ds(self.g * CHUNK + j * 16, 16)]
                pltpu.make_async_copy(
                    table_hbm.at[ivec],
                    rows_v.at[self.b].at[pl.ds(j * 16, 16)],
                    gsem.at[self.b],
                ).start()

        def wait(self):
            pltpu.make_async_copy(
                table_hbm.at[pl.ds(0, CHUNK)],
                rows_v.at[self.b],
                gsem.at[self.b],
            ).wait()

    def gather(g, b):
        return _G(g, b)

    def out_copy(g, b):
        return pltpu.make_async_copy(
            rows_v.at[b],
            out_hbm.at[pl.ds(base + g * CHUNK, CHUNK)],
            osem.at[b],
        )

    for b in range(LOOK):
        gather(b, b).start()

    for b in range(NBUF):
        g = b
        gather(g, b).wait()
        out_copy(g, b).start()
        h = g + LOOK
        hb = h % NBUF
        if h >= NBUF:
            out_copy(h - NBUF, hb).wait()
        gather(h, hb).start()

    def round_body(i, carry):
        for b in range(NBUF):
            g = i * NBUF + b
            hb = (b + LOOK) % NBUF
            gather(g, b).wait()
            out_copy(g, b).start()
            out_copy(g - (NBUF - LOOK), hb).wait()
            gather(g + LOOK, hb).start()
        return carry

    lax.fori_loop(1, NROUND - 1, round_body, 0)

    for b in range(NBUF):
        g = (NROUND - 1) * NBUF + b
        gather(g, b).wait()
        out_copy(g, b).start()
        h = g + LOOK
        if h < NCHUNK:
            hb = h % NBUF
            out_copy(g - (NBUF - LOOK), hb).wait()
            gather(h, hb).start()

    for b in range(NBUF):
        g = (NROUND - 1) * NBUF + b
        out_copy(g, b).wait()


@jax.jit
def kernel(x, emb):
    mesh = plsc.VectorSubcoreMesh(core_axis_name="c", subcore_axis_name="s")

    embT = jnp.swapaxes(emb, 0, 1)
    tail = jnp.pad(emb[999936:, :], ((0, 0), (0, 64)))

    tab_pairs = pl.kernel(
        _transpose_kernel,
        out_type=jax.ShapeDtypeStruct((N_ROWS // 2, 128), jnp.float32),
        mesh=mesh,
        scratch_types=[
            pltpu.VMEM((64, 128), jnp.float32),
            pltpu.VMEM((64, 128), jnp.float32),
            pltpu.SemaphoreType.DMA,
            pltpu.SemaphoreType.DMA,
        ],
        compiler_params=pltpu.CompilerParams(
            use_tc_tiling_on_sc=True, needs_layout_passes=False
        ),
    )(embT, tail)

    tab = tab_pairs.reshape(N_ROWS, D_EMBED)
    idx = x.reshape(-1).astype(jnp.int32)

    out = pl.kernel(
        _gather_kernel,
        out_type=jax.ShapeDtypeStruct((B_TOTAL, D_EMBED), jnp.float32),
        mesh=mesh,
        scratch_types=[
            pltpu.VMEM((B_PER_W,), jnp.int32),
            pltpu.VMEM((NBUF, CHUNK, D_EMBED), jnp.float32),
            pltpu.SemaphoreType.DMA((NBUF,)),
            pltpu.SemaphoreType.DMA((NBUF,)),
        ],
        compiler_params=pltpu.CompilerParams(use_tc_tiling_on_sc=False),
    )(idx, tab)
    return out.reshape(x.shape[0], x.shape[1], D_EMBED)


# speedup vs baseline: 1.0594x; 1.0594x over previous
import jax
import jax.numpy as jnp
from jax import lax
from jax.experimental import pallas as pl
from jax.experimental.pallas import tpu as pltpu
from jax.experimental.pallas import tpu_sc as plsc

N_ROWS = 1000000
D_EMBED = 64
B_TOTAL = 1024 * 200
NW = 32

NBLK = 999936 // 128
TAIL_ROWS = N_ROWS - 999936
TAIL_PAIRS = TAIL_ROWS // 2

B_PER_W = B_TOTAL // NW
CHUNK = 128
NCHUNK = B_PER_W // CHUNK
NBUF = 5
LOOK = 3
NROUND = NCHUNK // NBUF
GPB = CHUNK // 16


def _transpose_kernel(embT_hbm, tail_hbm, tab_hbm, src_v, dst_v, gsem, wsem):
    wid = lax.axis_index("s") * 2 + lax.axis_index("c")
    start = 244 * wid + jnp.minimum(wid, 4)
    count = jnp.where(wid < 4, 245, 244)
    iota16 = lax.iota(jnp.int32, 16)

    def block_body(k, carry):
        ti = start + k
        c0 = ti * 128
        for td in range(8):
            pltpu.make_async_copy(
                embT_hbm.at[pl.ds(8 * td, 8), pl.ds(c0, 128)],
                src_v.at[pl.ds(8 * td, 8)],
                gsem,
            ).start()

        @pl.when(k > 0)
        def _():
            pltpu.make_async_copy(
                dst_v, tab_hbm.at[pl.ds(0, 64)], wsem
            ).wait()

        pltpu.make_async_copy(
            embT_hbm.at[pl.ds(0, 64), pl.ds(0, 128)], src_v, gsem
        ).wait()

        def col_body(i, c):
            r0 = i * 8
            for r2 in range(8):
                r = r0 + r2
                half = 64 * (r % 2)
                rvec = jnp.full((16,), r, jnp.int32)
                for j in range(4):
                    vals = plsc.load_gather(src_v, [16 * j + iota16, rvec])
                    dst_v[r // 2, pl.ds(half + 16 * j, 16)] = vals
            return c

        lax.fori_loop(0, 16, col_body, 0)
        pltpu.make_async_copy(
            dst_v, tab_hbm.at[pl.ds(64 * ti, 64)], wsem
        ).start()
        return carry

    lax.fori_loop(0, count, block_body, 0)
    pltpu.make_async_copy(dst_v, tab_hbm.at[pl.ds(0, 64)], wsem).wait()

    @pl.when(wid == NW - 1)
    def _tail():
        pltpu.sync_copy(tail_hbm, src_v)
        for q in range(TAIL_PAIRS):
            for j in range(4):
                dst_v[q, pl.ds(16 * j, 16)] = src_v[2 * q, pl.ds(16 * j, 16)]
                dst_v[q, pl.ds(64 + 16 * j, 16)] = src_v[
                    2 * q + 1, pl.ds(16 * j, 16)
                ]
        pltpu.sync_copy(
            dst_v.at[pl.ds(0, TAIL_PAIRS)],
            tab_hbm.at[pl.ds(NBLK * 64, TAIL_PAIRS)],
        )


def _gather_kernel(idx_hbm, table_hbm, out_hbm, idx_v, rows_v, gsem, osem):
    wid = lax.axis_index("s") * 2 + lax.axis_index("c")
    base = wid * B_PER_W
    pltpu.sync_copy(idx_hbm.at[pl.ds(base, B_PER_W)], idx_v)

    class _G:
        def __init__(self, g, b):
            self.g, self.b = g, b

        def start(self):
            for j in range(GPB):
                ivec = idx_v[pl.ds(self.g * CHUNK + j * 16, 16)]
                pltpu.make_async_copy(
                    table_hbm.at[ivec],
                    rows_v.at[self.b].at[pl.ds(j * 16, 16)],
                    gsem.at[self.b],
                ).start()

        def wait(self):
            pltpu.make_async_copy(
                table_hbm.at[pl.ds(0, CHUNK)],
                rows_v.at[self.b],
                gsem.at[self.b],
            ).wait()

    def gather(g, b):
        return _G(g, b)

    def out_copy(g, b):
        return pltpu.make_async_copy(
            rows_v.at[b],
            out_hbm.at[pl.ds(base + g * CHUNK, CHUNK)],
            osem.at[b],
        )

    for b in range(LOOK):
        gather(b, b).start()

    for b in range(NBUF):
        g = b
        gather(g, b).wait()
        out_copy(g, b).start()
        h = g + LOOK
        hb = h % NBUF
        if h >= NBUF:
            out_copy(h - NBUF, hb).wait()
        gather(h, hb).start()

    def round_body(i, carry):
        for b in range(NBUF):
            g = i * NBUF + b
            hb = (b + LOOK) % NBUF
            gather(g, b).wait()
            out_copy(g, b).start()
            out_copy(g - (NBUF - LOOK), hb).wait()
            gather(g + LOOK, hb).start()
        return carry

    lax.fori_loop(1, NROUND - 1, round_body, 0)

    for b in range(NBUF):
        g = (NROUND - 1) * NBUF + b
        gather(g, b).wait()
        out_copy(g, b).start()
        h = g + LOOK
        if h < NCHUNK:
            hb = h % NBUF
            out_copy(g - (NBUF - LOOK), hb).wait()
            gather(h, hb).start()

    for b in range(NBUF):
        g = (NROUND - 1) * NBUF + b
        out_copy(g, b).wait()


@jax.jit
def kernel(x, emb):
    mesh = plsc.VectorSubcoreMesh(core_axis_name="c", subcore_axis_name="s")

    embT = jnp.swapaxes(emb, 0, 1)
    tail = jnp.pad(emb[999936:, :], ((0, 0), (0, 64)))

    tab_pairs = pl.kernel(
        _transpose_kernel,
        out_type=jax.ShapeDtypeStruct((N_ROWS // 2, 128), jnp.float32),
        mesh=mesh,
        scratch_types=[
            pltpu.VMEM((64, 128), jnp.float32),
            pltpu.VMEM((64, 128), jnp.float32),
            pltpu.SemaphoreType.DMA,
            pltpu.SemaphoreType.DMA,
        ],
        compiler_params=pltpu.CompilerParams(
            use_tc_tiling_on_sc=True, needs_layout_passes=False
        ),
    )(embT, tail)

    tab = tab_pairs.reshape(N_ROWS, D_EMBED)
    idx = x.reshape(-1).astype(jnp.int32)

    out = pl.kernel(
        _gather_kernel,
        out_type=jax.ShapeDtypeStruct((B_TOTAL, D_EMBED), jnp.float32),
        mesh=mesh,
        scratch_types=[
            pltpu.VMEM((B_PER_W,), jnp.int32),
            pltpu.VMEM((NBUF, CHUNK, D_EMBED), jnp.float32),
            pltpu.SemaphoreType.DMA((NBUF,)),
            pltpu.SemaphoreType.DMA((NBUF,)),
        ],
        compiler_params=pltpu.CompilerParams(use_tc_tiling_on_sc=False),
    )(idx, tab)
    return out.reshape(x.shape[0], x.shape[1], D_EMBED)


# speedup vs baseline: 1.6650x; 1.5716x over previous
import jax
import jax.numpy as jnp
from jax import lax
from jax.experimental import pallas as pl
from jax.experimental.pallas import tpu as pltpu
from jax.experimental.pallas import tpu_sc as plsc

N_ROWS = 1000000
D_EMBED = 64
B_TOTAL = 1024 * 200
NW = 32

NBLK = 999936 // 128
TAIL_ROWS = N_ROWS - 999936
TAIL_PAIRS = TAIL_ROWS // 2

B_PER_W = B_TOTAL // NW
CHUNK = 128
NCHUNK = B_PER_W // CHUNK
NBUF = 5
LOOK = 3
NROUND = NCHUNK // NBUF
GPB = CHUNK // 16


def _transpose_kernel(embT_hbm, tail_hbm, tab_hbm, src_v, dst_v, gsem, wsem):
    wid = lax.axis_index("s") * 2 + lax.axis_index("c")
    start = 244 * wid + jnp.minimum(wid, 4)
    count = jnp.where(wid < 4, 245, 244)
    iota16 = lax.iota(jnp.int32, 16)

    def block_body(k, carry):
        ti = start + k
        c0 = ti * 128
        for td in range(8):
            pltpu.make_async_copy(
                embT_hbm.at[pl.ds(8 * td, 8), pl.ds(c0, 128)],
                src_v.at[pl.ds(8 * td, 8)],
                gsem,
            ).start()

        @pl.when(k > 0)
        def _():
            pltpu.make_async_copy(
                dst_v, tab_hbm.at[pl.ds(0, 64)], wsem
            ).wait()

        pltpu.make_async_copy(
            embT_hbm.at[pl.ds(0, 64), pl.ds(0, 128)], src_v, gsem
        ).wait()

        @plsc.parallel_loop(0, 128, 2, unroll=4)
        def col_body(r0):
            for r2 in range(2):
                r = r0 + r2
                rvec = jnp.full((16,), r, jnp.int32)
                for j in range(4):
                    vals = plsc.load_gather(src_v, [16 * j + iota16, rvec])
                    dst_v[r0 // 2, pl.ds(64 * r2 + 16 * j, 16)] = vals
        pltpu.make_async_copy(
            dst_v, tab_hbm.at[pl.ds(64 * ti, 64)], wsem
        ).start()
        return carry

    lax.fori_loop(0, count, block_body, 0)
    pltpu.make_async_copy(dst_v, tab_hbm.at[pl.ds(0, 64)], wsem).wait()

    @pl.when(wid == NW - 1)
    def _tail():
        pltpu.sync_copy(tail_hbm, src_v)
        for q in range(TAIL_PAIRS):
            for j in range(4):
                dst_v[q, pl.ds(16 * j, 16)] = src_v[2 * q, pl.ds(16 * j, 16)]
                dst_v[q, pl.ds(64 + 16 * j, 16)] = src_v[
                    2 * q + 1, pl.ds(16 * j, 16)
                ]
        pltpu.sync_copy(
            dst_v.at[pl.ds(0, TAIL_PAIRS)],
            tab_hbm.at[pl.ds(NBLK * 64, TAIL_PAIRS)],
        )


def _gather_kernel(idx_hbm, table_hbm, out_hbm, idx_v, rows_v, gsem, osem):
    wid = lax.axis_index("s") * 2 + lax.axis_index("c")
    base = wid * B_PER_W
    pltpu.sync_copy(idx_hbm.at[pl.ds(base, B_PER_W)], idx_v)

    class _G:
        def __init__(self, g, b):
            self.g, self.b = g, b

        def start(self):
            for j in range(GPB):
                ivec = idx_v[pl.ds(self.g * CHUNK + j * 16, 16)]
                pltpu.make_async_copy(
                    table_hbm.at[ivec],
                    rows_v.at[self.b].at[pl.ds(j * 16, 16)],
                    gsem.at[self.b],
                ).start()

        def wait(self):
            pltpu.make_async_copy(
                table_hbm.at[pl.ds(0, CHUNK)],
                rows_v.at[self.b],
                gsem.at[self.b],
            ).wait()

    def gather(g, b):
        return _G(g, b)

    def out_copy(g, b):
        return pltpu.make_async_copy(
            rows_v.at[b],
            out_hbm.at[pl.ds(base + g * CHUNK, CHUNK)],
            osem.at[b],
        )

    for b in range(LOOK):
        gather(b, b).start()

    for b in range(NBUF):
        g = b
        gather(g, b).wait()
        out_copy(g, b).start()
        h = g + LOOK
        hb = h % NBUF
        if h >= NBUF:
            out_copy(h - NBUF, hb).wait()
        gather(h, hb).start()

    def round_body(i, carry):
        for b in range(NBUF):
            g = i * NBUF + b
            hb = (b + LOOK) % NBUF
            gather(g, b).wait()
            out_copy(g, b).start()
            out_copy(g - (NBUF - LOOK), hb).wait()
            gather(g + LOOK, hb).start()
        return carry

    lax.fori_loop(1, NROUND - 1, round_body, 0)

    for b in range(NBUF):
        g = (NROUND - 1) * NBUF + b
        gather(g, b).wait()
        out_copy(g, b).start()
        h = g + LOOK
        if h < NCHUNK:
            hb = h % NBUF
            out_copy(g - (NBUF - LOOK), hb).wait()
            gather(h, hb).start()

    for b in range(NBUF):
        g = (NROUND - 1) * NBUF + b
        out_copy(g, b).wait()


@jax.jit
def kernel(x, emb):
    mesh = plsc.VectorSubcoreMesh(core_axis_name="c", subcore_axis_name="s")

    embT = jnp.swapaxes(emb, 0, 1)
    tail = jnp.pad(emb[999936:, :], ((0, 0), (0, 64)))

    tab_pairs = pl.kernel(
        _transpose_kernel,
        out_type=jax.ShapeDtypeStruct((N_ROWS // 2, 128), jnp.float32),
        mesh=mesh,
        scratch_types=[
            pltpu.VMEM((64, 128), jnp.float32),
            pltpu.VMEM((64, 128), jnp.float32),
            pltpu.SemaphoreType.DMA,
            pltpu.SemaphoreType.DMA,
        ],
        compiler_params=pltpu.CompilerParams(
            use_tc_tiling_on_sc=True, needs_layout_passes=False
        ),
    )(embT, tail)

    tab = tab_pairs.reshape(N_ROWS, D_EMBED)
    idx = x.reshape(-1).astype(jnp.int32)

    out = pl.kernel(
        _gather_kernel,
        out_type=jax.ShapeDtypeStruct((B_TOTAL, D_EMBED), jnp.float32),
        mesh=mesh,
        scratch_types=[
            pltpu.VMEM((B_PER_W,), jnp.int32),
            pltpu.VMEM((NBUF, CHUNK, D_EMBED), jnp.float32),
            pltpu.SemaphoreType.DMA((NBUF,)),
            pltpu.SemaphoreType.DMA((NBUF,)),
        ],
        compiler_params=pltpu.CompilerParams(use_tc_tiling_on_sc=False),
    )(idx, tab)
    return out.reshape(x.shape[0], x.shape[1], D_EMBED)


# speedup vs baseline: 2.0906x; 1.2556x over previous
import jax
import jax.numpy as jnp
from jax import lax
from jax.experimental import pallas as pl
from jax.experimental.pallas import tpu as pltpu
from jax.experimental.pallas import tpu_sc as plsc

N_ROWS = 1000000
D_EMBED = 64
B_TOTAL = 1024 * 200
NW = 32

NBLK = 999936 // 128
TAIL_ROWS = N_ROWS - 999936
TAIL_PAIRS = TAIL_ROWS // 2

B_PER_W = B_TOTAL // NW
CHUNK = 128
NCHUNK = B_PER_W // CHUNK
NBUF = 5
LOOK = 3
NROUND = NCHUNK // NBUF
GPB = CHUNK // 16


def _transpose_kernel(embT_hbm, tail_hbm, tab_hbm, src_v, dst_v, gsem, wsem):
    wid = lax.axis_index("s") * 2 + lax.axis_index("c")
    start = 244 * wid + jnp.minimum(wid, 4)
    count = jnp.where(wid < 4, 245, 244)
    iota16 = lax.iota(jnp.int32, 16)

    def fire_reads(k, b):
        c0 = (start + k) * 128
        for td in range(8):
            pltpu.make_async_copy(
                embT_hbm.at[pl.ds(8 * td, 8), pl.ds(c0, 128)],
                src_v.at[b].at[pl.ds(8 * td, 8)],
                gsem.at[b],
            ).start()

    def drain_reads(b):
        pltpu.make_async_copy(
            embT_hbm.at[pl.ds(0, 64), pl.ds(0, 128)], src_v.at[b], gsem.at[b]
        ).wait()

    def wait_write(b):
        pltpu.make_async_copy(
            dst_v.at[b], tab_hbm.at[pl.ds(0, 64)], wsem.at[b]
        ).wait()

    fire_reads(0, 0)

    def block_pair(i, carry):
        for b in range(2):
            k = 2 * i + b

            @pl.when(k + 1 < count)
            def _():
                fire_reads(k + 1, 1 - b)

            @pl.when(k < count)
            def _():
                drain_reads(b)

                @pl.when(k >= 2)
                def _():
                    wait_write(b)

                src_b, dst_b = src_v.at[b], dst_v.at[b]

                @plsc.parallel_loop(0, 128, 2, unroll=8)
                def col_body(r0):
                    for r2 in range(2):
                        rvec = jnp.full((16,), r0 + r2, jnp.int32)
                        for j in range(4):
                            vals = plsc.load_gather(
                                src_b, [16 * j + iota16, rvec]
                            )
                            dst_b[r0 // 2, pl.ds(64 * r2 + 16 * j, 16)] = vals

                pltpu.make_async_copy(
                    dst_b, tab_hbm.at[pl.ds(64 * (start + k), 64)], wsem.at[b]
                ).start()
        return carry

    lax.fori_loop(0, 123, block_pair, 0)
    wait_write(0)
    wait_write(1)

    @pl.when(wid == NW - 1)
    def _tail():
        t_src, t_dst = src_v.at[0], dst_v.at[0]
        pltpu.sync_copy(tail_hbm, t_src)
        for q in range(TAIL_PAIRS):
            for j in range(4):
                t_dst[q, pl.ds(16 * j, 16)] = t_src[2 * q, pl.ds(16 * j, 16)]
                t_dst[q, pl.ds(64 + 16 * j, 16)] = t_src[
                    2 * q + 1, pl.ds(16 * j, 16)
                ]
        pltpu.sync_copy(
            t_dst.at[pl.ds(0, TAIL_PAIRS)],
            tab_hbm.at[pl.ds(NBLK * 64, TAIL_PAIRS)],
        )


def _gather_kernel(idx_hbm, table_hbm, out_hbm, idx_v, rows_v, gsem, osem):
    wid = lax.axis_index("s") * 2 + lax.axis_index("c")
    base = wid * B_PER_W
    pltpu.sync_copy(idx_hbm.at[pl.ds(base, B_PER_W)], idx_v)

    class _G:
        def __init__(self, g, b):
            self.g, self.b = g, b

        def start(self):
            for j in range(GPB):
                ivec = idx_v[pl.ds(self.g * CHUNK + j * 16, 16)]
                pltpu.make_async_copy(
                    table_hbm.at[ivec],
                    rows_v.at[self.b].at[pl.ds(j * 16, 16)],
                    gsem.at[self.b],
                ).start()

        def wait(self):
            pltpu.make_async_copy(
                table_hbm.at[pl.ds(0, CHUNK)],
                rows_v.at[self.b],
                gsem.at[self.b],
            ).wait()

    def gather(g, b):
        return _G(g, b)

    def out_copy(g, b):
        return pltpu.make_async_copy(
            rows_v.at[b],
            out_hbm.at[pl.ds(base + g * CHUNK, CHUNK)],
            osem.at[b],
        )

    for b in range(LOOK):
        gather(b, b).start()

    for b in range(NBUF):
        g = b
        gather(g, b).wait()
        out_copy(g, b).start()
        h = g + LOOK
        hb = h % NBUF
        if h >= NBUF:
            out_copy(h - NBUF, hb).wait()
        gather(h, hb).start()

    def round_body(i, carry):
        for b in range(NBUF):
            g = i * NBUF + b
            hb = (b + LOOK) % NBUF
            gather(g, b).wait()
            out_copy(g, b).start()
            out_copy(g - (NBUF - LOOK), hb).wait()
            gather(g + LOOK, hb).start()
        return carry

    lax.fori_loop(1, NROUND - 1, round_body, 0)

    for b in range(NBUF):
        g = (NROUND - 1) * NBUF + b
        gather(g, b).wait()
        out_copy(g, b).start()
        h = g + LOOK
        if h < NCHUNK:
            hb = h % NBUF
            out_copy(g - (NBUF - LOOK), hb).wait()
            gather(h, hb).start()

    for b in range(NBUF):
        g = (NROUND - 1) * NBUF + b
        out_copy(g, b).wait()


@jax.jit
def kernel(x, emb):
    mesh = plsc.VectorSubcoreMesh(core_axis_name="c", subcore_axis_name="s")

    embT = jnp.swapaxes(emb, 0, 1)
    tail = jnp.pad(emb[999936:, :], ((0, 0), (0, 64)))

    tab_pairs = pl.kernel(
        _transpose_kernel,
        out_type=jax.ShapeDtypeStruct((N_ROWS // 2, 128), jnp.float32),
        mesh=mesh,
        scratch_types=[
            pltpu.VMEM((2, 64, 128), jnp.float32),
            pltpu.VMEM((2, 64, 128), jnp.float32),
            pltpu.SemaphoreType.DMA((2,)),
            pltpu.SemaphoreType.DMA((2,)),
        ],
        compiler_params=pltpu.CompilerParams(
            use_tc_tiling_on_sc=True, needs_layout_passes=False
        ),
    )(embT, tail)

    tab = tab_pairs.reshape(N_ROWS, D_EMBED)
    idx = x.reshape(-1).astype(jnp.int32)

    out = pl.kernel(
        _gather_kernel,
        out_type=jax.ShapeDtypeStruct((B_TOTAL, D_EMBED), jnp.float32),
        mesh=mesh,
        scratch_types=[
            pltpu.VMEM((B_PER_W,), jnp.int32),
            pltpu.VMEM((NBUF, CHUNK, D_EMBED), jnp.float32),
            pltpu.SemaphoreType.DMA((NBUF,)),
            pltpu.SemaphoreType.DMA((NBUF,)),
        ],
        compiler_params=pltpu.CompilerParams(use_tc_tiling_on_sc=False),
    )(idx, tab)
    return out.reshape(x.shape[0], x.shape[1], D_EMBED)


# speedup vs baseline: 2.7847x; 1.3320x over previous
import jax
import jax.numpy as jnp
from jax import lax
from jax.experimental import pallas as pl
from jax.experimental.pallas import tpu as pltpu
from jax.experimental.pallas import tpu_sc as plsc

B_TOTAL = 1024 * 200
D_EMBED = 64
D_PAD = 128
NW = 32
B_PER_W = B_TOTAL // NW
CHUNK = 128
NCHUNK = B_PER_W // CHUNK
NBUF = 5
LOOK = 3
NROUND = NCHUNK // NBUF
GPB = CHUNK // 16


def _gather_kernel(idx_hbm, table_hbm, out_hbm, idx_v, rows_v, gsem, osem):
    wid = lax.axis_index("s") * 2 + lax.axis_index("c")
    base = wid * B_PER_W
    pltpu.sync_copy(idx_hbm.at[pl.ds(base, B_PER_W)], idx_v)

    class _G:
        def __init__(self, g, b):
            self.g, self.b = g, b

        def start(self):
            for j in range(GPB):
                ivec = idx_v[pl.ds(self.g * CHUNK + j * 16, 16)]
                pltpu.make_async_copy(
                    table_hbm.at[ivec],
                    rows_v.at[self.b].at[pl.ds(j * 16, 16)],
                    gsem.at[self.b],
                ).start()

        def wait(self):
            pltpu.make_async_copy(
                table_hbm.at[pl.ds(0, CHUNK)],
                rows_v.at[self.b],
                gsem.at[self.b],
            ).wait()

    def gather(g, b):
        return _G(g, b)

    def out_copy(g, b):
        return pltpu.make_async_copy(
            rows_v.at[b].at[:, pl.ds(0, D_EMBED)],
            out_hbm.at[pl.ds(base + g * CHUNK, CHUNK)],
            osem.at[b],
        )

    for b in range(LOOK):
        gather(b, b).start()

    for b in range(NBUF):
        g = b
        gather(g, b).wait()
        out_copy(g, b).start()
        h = g + LOOK
        hb = h % NBUF
        if h >= NBUF:
            out_copy(h - NBUF, hb).wait()
        gather(h, hb).start()

    def round_body(i, carry):
        for b in range(NBUF):
            g = i * NBUF + b
            hb = (b + LOOK) % NBUF
            gather(g, b).wait()
            out_copy(g, b).start()
            out_copy(g - (NBUF - LOOK), hb).wait()
            gather(g + LOOK, hb).start()
        return carry

    lax.fori_loop(1, NROUND - 1, round_body, 0)

    for b in range(NBUF):
        g = (NROUND - 1) * NBUF + b
        gather(g, b).wait()
        out_copy(g, b).start()
        h = g + LOOK
        if h < NCHUNK:
            hb = h % NBUF
            out_copy(g - (NBUF - LOOK), hb).wait()
            gather(h, hb).start()

    for b in range(NBUF):
        g = (NROUND - 1) * NBUF + b
        out_copy(g, b).wait()


@jax.jit
def kernel(x, emb):
    idx = x.reshape(-1).astype(jnp.int32)
    embp = jnp.pad(emb, ((0, 0), (0, D_PAD - D_EMBED)))
    mesh = plsc.VectorSubcoreMesh(core_axis_name="c", subcore_axis_name="s")
    out = pl.kernel(
        _gather_kernel,
        out_type=jax.ShapeDtypeStruct((B_TOTAL, D_EMBED), jnp.float32),
        mesh=mesh,
        scratch_types=[
            pltpu.VMEM((B_PER_W,), jnp.int32),
            pltpu.VMEM((NBUF, CHUNK, D_PAD), jnp.float32),
            pltpu.SemaphoreType.DMA((NBUF,)),
            pltpu.SemaphoreType.DMA((NBUF,)),
        ],
        compiler_params=pltpu.CompilerParams(use_tc_tiling_on_sc=False),
    )(idx, embp)
    return out.reshape(x.shape[0], x.shape[1], D_EMBED)
